# baseline (speedup 1.0000x reference)
import functools

import jax
import jax.numpy as jnp
from jax import lax
from jax.experimental import pallas as pl
from jax.experimental.pallas import tpu as pltpu
from jax.experimental.pallas import tpu_sc as plsc

N = 10000
E = 320000
D_IN = 128
D_HID = 128
C = 40
CP = 128

NC, NS = 2, 16
NW = NC * NS
E_W = E // NW
K = 128
NB = 2
E_WP = 10240
PH = 5
PCH = E_WP // (PH * K)
NP = N + 8 * NW
DEG_CHUNK = 1000


def _make_sc_agg(D, with_deg):
  mesh = plsc.VectorSubcoreMesh(
      core_axis_name="c", subcore_axis_name="s",
      num_cores=NC, num_subcores=NS)

  out_type = [jax.ShapeDtypeStruct((NC, N, D), jnp.float32)]
  scratch = [
      pltpu.VMEM((PCH, K), jnp.int32),
      pltpu.VMEM((PCH, K), jnp.int32),
  ] + [pltpu.VMEM((K, D), jnp.float32) for _ in range(NB)] + [
      pltpu.VMEM_SHARED((NP, D), jnp.float32),
  ] + [pltpu.SemaphoreType.DMA for _ in range(2 * NB)]
  if with_deg:
    out_type.append(jax.ShapeDtypeStruct((NC * N,), jnp.float32))
    scratch += [
        pltpu.VMEM((K,), jnp.float32),
        pltpu.VMEM_SHARED((NP,), jnp.float32),
        pltpu.VMEM((1008,), jnp.float32),
    ] + [pltpu.SemaphoreType.DMA for _ in range(NB)]

  def body(*refs):
    x_hbm, src_hbm, dst_hbm, z2_hbm = refs[:4]
    nout = 2 if with_deg else 1
    agg_out = refs[4]
    k = 4 + nout
    srcv, dstv = refs[k], refs[k + 1]
    rows = refs[k + 2:k + 2 + NB]
    acc_sh = refs[k + 2 + NB]
    gsem = refs[k + 3 + NB:k + 3 + 2 * NB]
    ssem = refs[k + 3 + 2 * NB:k + 3 + 3 * NB]
    if with_deg:
      deg_out = refs[5]
      onesv, deg_sh, degbuf = refs[k + 3 + 3 * NB:k + 6 + 3 * NB]
      dsem = refs[k + 6 + 3 * NB:k + 6 + 4 * NB]

    c = lax.axis_index("c")
    s = lax.axis_index("s")
    wid = c * NS + s

    @pl.when(s < N // DEG_CHUNK)
    def _():
      pltpu.sync_copy(z2_hbm.at[pl.ds(s * DEG_CHUNK, DEG_CHUNK)],
                      acc_sh.at[pl.ds(s * DEG_CHUNK, DEG_CHUNK)])
    if with_deg:
      for i in range(1008 // 16):
        degbuf[pl.ds(i * 16, 16)] = jnp.zeros((16,), jnp.float32)
      @pl.when(s < N // DEG_CHUNK)
      def _():
        pltpu.sync_copy(degbuf.at[pl.ds(0, DEG_CHUNK)],
                        deg_sh.at[pl.ds(s * DEG_CHUNK, DEG_CHUNK)])
      for i in range(K // 16):
        onesv[pl.ds(i * 16, 16)] = jnp.full((16,), 1.0, jnp.float32)

    plsc.subcore_barrier()

    def fire_gather(jj, b):
      pltpu.async_copy(x_hbm.at[srcv.at[jj]], rows[b], gsem[b])

    def wait_gather(b):
      pltpu.make_async_copy(x_hbm.at[srcv.at[0]], rows[b], gsem[b]).wait()

    def fire_scatter(jj, b):
      pltpu.async_copy(rows[b], acc_sh.at[dstv.at[jj]], ssem[b], add=True)

    def wait_scatter(b):
      pltpu.make_async_copy(rows[b], acc_sh.at[dstv.at[0]], ssem[b]).wait()

    if with_deg:
      def fire_deg(jj, b):
        pltpu.async_copy(onesv, deg_sh.at[dstv.at[jj]],
                         dsem[b], add=True)

      def wait_deg(b):
        pltpu.make_async_copy(onesv, deg_sh.at[dstv.at[0]],
                              dsem[b]).wait()

    for p in range(PH):
      pltpu.sync_copy(src_hbm.at[wid * PH + p], srcv)
      pltpu.sync_copy(dst_hbm.at[wid * PH + p], dstv)

      fire_gather(0, 0)

      def tbody(t, carry):
        for u in range(2):
          i = 2 * t + u
          b = u
          bn = 1 - u
          wait_gather(b)
          fire_scatter(i, b)
          if with_deg:
            fire_deg(i, b)
          if u == 0:
            @pl.when(i >= 1)
            def _():
              wait_scatter(bn)
              if with_deg:
                wait_deg(bn)
            fire_gather(i + 1, bn)
          else:
            wait_scatter(bn)
            if with_deg:
              wait_deg(bn)
            @pl.when(i + 1 < PCH)
            def _():
              fire_gather(i + 1, bn)
        return carry

      lax.fori_loop(0, PCH // 2, tbody, 0)

      wait_scatter((PCH - 1) % 2)
      if with_deg:
        wait_deg((PCH - 1) % 2)

    plsc.subcore_barrier()

    @pl.when(s < N // DEG_CHUNK)
    def _():
      pltpu.sync_copy(acc_sh.at[pl.ds(s * DEG_CHUNK, DEG_CHUNK)],
                      agg_out.at[c, pl.ds(s * DEG_CHUNK, DEG_CHUNK)])
    if with_deg:
      @pl.when(s < N // DEG_CHUNK)
      def _():
        pltpu.sync_copy(deg_sh.at[pl.ds(s * DEG_CHUNK, DEG_CHUNK)],
                        degbuf.at[pl.ds(0, DEG_CHUNK)])
        pltpu.sync_copy(degbuf.at[pl.ds(0, DEG_CHUNK)],
                        deg_out.at[pl.ds(c * N + s * DEG_CHUNK, DEG_CHUNK)])

  return pl.kernel(body, out_type=out_type, mesh=mesh,
                   scratch_types=scratch)


_sc_agg_deg = _make_sc_agg(D_IN, with_deg=True)
_sc_agg_p = _make_sc_agg(CP, with_deg=False)

R = 1000


def _tc1_body(a0, a1, d0, d1, x, w1, b1, w2, p_out):
  num = a0[0] + a1[0] + x[...]
  den = d0[0] + d1[0] + 1.0
  h = num / den
  h = jnp.maximum(jnp.dot(h, w1[...], preferred_element_type=jnp.float32)
                  + b1[...], 0.0)
  p_out[...] = jnp.dot(h, w2[...], preferred_element_type=jnp.float32)


def _tc2_body(g0, g1, d0, d1, p, b2, out):
  den = d0[0] + d1[0] + 1.0
  t = (g0[0] + g1[0] + p[...]) / den + b2[...]
  out[...] = t[:, :C]


def kernel(x, edge_index, W1, b1, W2, b2):
  pad = E_WP - E_W
  src2 = edge_index[0].astype(jnp.int32).reshape(NW, E_W)
  dst2 = edge_index[1].astype(jnp.int32).reshape(NW, E_W)
  src3 = jnp.pad(src2, ((0, 0), (0, pad))).reshape(NW * PH, PCH, K)
  dpad = (N + 8 * jnp.arange(NW, dtype=jnp.int32)[:, None]
          + (jnp.arange(pad, dtype=jnp.int32)[None, :] % 8))
  dst3 = jnp.concatenate([dst2, dpad], axis=1).reshape(NW * PH, PCH, K)
  z2 = jnp.zeros((N, D_IN), jnp.float32)
  w2p = jnp.pad(W2, ((0, 0), (0, CP - C)))
  b2p = jnp.pad(b2, (0, CP - C)).reshape(1, CP)

  aggp, degp = _sc_agg_deg(x, src3, dst3, z2)
  degp3 = degp.reshape(NC, N, 1)

  grid = (N // R,)
  p = pl.pallas_call(
      _tc1_body,
      grid=grid,
      in_specs=[
          pl.BlockSpec((1, R, D_IN), lambda i: (0, i, 0)),
          pl.BlockSpec((1, R, D_IN), lambda i: (1, i, 0)),
          pl.BlockSpec((1, R, 1), lambda i: (0, i, 0)),
          pl.BlockSpec((1, R, 1), lambda i: (1, i, 0)),
          pl.BlockSpec((R, D_IN), lambda i: (i, 0)),
          pl.BlockSpec((D_IN, D_HID), lambda i: (0, 0)),
          pl.BlockSpec((1, D_HID), lambda i: (0, 0)),
          pl.BlockSpec((D_HID, CP), lambda i: (0, 0)),
      ],
      out_specs=pl.BlockSpec((R, CP), lambda i: (i, 0)),
      out_shape=jax.ShapeDtypeStruct((N, CP), jnp.float32),
  )(aggp, aggp, degp3, degp3, x, W1, b1.reshape(1, D_HID), w2p)

  (gp,) = _sc_agg_p(p, src3, dst3, z2)

  out = pl.pallas_call(
      _tc2_body,
      grid=grid,
      in_specs=[
          pl.BlockSpec((1, R, CP), lambda i: (0, i, 0)),
          pl.BlockSpec((1, R, CP), lambda i: (1, i, 0)),
          pl.BlockSpec((1, R, 1), lambda i: (0, i, 0)),
          pl.BlockSpec((1, R, 1), lambda i: (1, i, 0)),
          pl.BlockSpec((R, CP), lambda i: (i, 0)),
          pl.BlockSpec((1, CP), lambda i: (0, 0)),
      ],
      out_specs=pl.BlockSpec((R, C), lambda i: (i, 0)),
      out_shape=jax.ShapeDtypeStruct((N, C), jnp.float32),
  )(gp, gp, degp3, degp3, p, b2p)

  return out


# speedup vs baseline: 2.2734x; 2.2734x over previous
import functools

import jax
import jax.numpy as jnp
from jax import lax
from jax.experimental import pallas as pl
from jax.experimental.pallas import tpu as pltpu
from jax.experimental.pallas import tpu_sc as plsc

N = 10000
E = 320000
D_IN = 128
D_HID = 128
C = 40
CP = 128

NC, NS = 2, 16
NW = NC * NS
E_W = E // NW
K = 80
NB = 2
PH = 5
PCH = E_W // (PH * K)
NP = N
DEG_CHUNK = 1000


def _make_sc_agg(D, with_deg):
  mesh = plsc.VectorSubcoreMesh(
      core_axis_name="c", subcore_axis_name="s",
      num_cores=NC, num_subcores=NS)

  out_type = [jax.ShapeDtypeStruct((NC, N, D), jnp.float32)]
  scratch = [
      pltpu.VMEM((PCH, K), jnp.int32),
      pltpu.VMEM((PCH, K), jnp.int32),
  ] + [pltpu.VMEM((K, D), jnp.float32) for _ in range(NB)] + [
      pltpu.VMEM_SHARED((NP, D), jnp.float32),
  ] + [pltpu.SemaphoreType.DMA for _ in range(2 * NB)]
  if with_deg:
    out_type.append(jax.ShapeDtypeStruct((NC * N,), jnp.float32))
    scratch += [
        pltpu.VMEM((K,), jnp.float32),
        pltpu.VMEM_SHARED((NP,), jnp.float32),
        pltpu.VMEM((1008,), jnp.float32),
    ] + [pltpu.SemaphoreType.DMA for _ in range(NB)]

  def body(*refs):
    x_hbm, src_hbm, dst_hbm, z2_hbm = refs[:4]
    nout = 2 if with_deg else 1
    agg_out = refs[4]
    k = 4 + nout
    srcv, dstv = refs[k], refs[k + 1]
    rows = refs[k + 2:k + 2 + NB]
    acc_sh = refs[k + 2 + NB]
    gsem = refs[k + 3 + NB:k + 3 + 2 * NB]
    ssem = refs[k + 3 + 2 * NB:k + 3 + 3 * NB]
    if with_deg:
      deg_out = refs[5]
      onesv, deg_sh, degbuf = refs[k + 3 + 3 * NB:k + 6 + 3 * NB]
      dsem = refs[k + 6 + 3 * NB:k + 6 + 4 * NB]

    c = lax.axis_index("c")
    s = lax.axis_index("s")
    wid = c * NS + s

    @pl.when(s < N // DEG_CHUNK)
    def _():
      pltpu.sync_copy(z2_hbm.at[pl.ds(s * DEG_CHUNK, DEG_CHUNK)],
                      acc_sh.at[pl.ds(s * DEG_CHUNK, DEG_CHUNK)])
    if with_deg:
      for i in range(1008 // 16):
        degbuf[pl.ds(i * 16, 16)] = jnp.zeros((16,), jnp.float32)
      @pl.when(s < N // DEG_CHUNK)
      def _():
        pltpu.sync_copy(degbuf.at[pl.ds(0, DEG_CHUNK)],
                        deg_sh.at[pl.ds(s * DEG_CHUNK, DEG_CHUNK)])
      for i in range(K // 16):
        onesv[pl.ds(i * 16, 16)] = jnp.full((16,), 1.0, jnp.float32)

    plsc.subcore_barrier()

    def fire_gather(jj, b):
      pltpu.async_copy(x_hbm.at[srcv.at[jj]], rows[b], gsem[b])

    def wait_gather(b):
      pltpu.make_async_copy(x_hbm.at[srcv.at[0]], rows[b], gsem[b]).wait()

    def fire_scatter(jj, b):
      pltpu.async_copy(rows[b], acc_sh.at[dstv.at[jj]], ssem[b], add=True)

    def wait_scatter(b):
      pltpu.make_async_copy(rows[b], acc_sh.at[dstv.at[0]], ssem[b]).wait()

    if with_deg:
      def fire_deg(jj, b):
        pltpu.async_copy(onesv, deg_sh.at[dstv.at[jj]],
                         dsem[b], add=True)

      def wait_deg(b):
        pltpu.make_async_copy(onesv, deg_sh.at[dstv.at[0]],
                              dsem[b]).wait()

    for p in range(PH):
      pltpu.sync_copy(src_hbm.at[wid * PH + p], srcv)
      pltpu.sync_copy(dst_hbm.at[wid * PH + p], dstv)

      fire_gather(0, 0)

      def tbody(t, carry):
        for u in range(2):
          i = 2 * t + u
          b = u
          bn = 1 - u
          wait_gather(b)
          fire_scatter(i, b)
          if with_deg:
            fire_deg(i, b)
          if u == 0:
            @pl.when(i >= 1)
            def _():
              wait_scatter(bn)
              if with_deg:
                wait_deg(bn)
            fire_gather(i + 1, bn)
          else:
            wait_scatter(bn)
            if with_deg:
              wait_deg(bn)
            fire_gather(i + 1, bn)
        return carry

      lax.fori_loop(0, PCH // 2, tbody, 0)

      wait_gather(0)
      fire_scatter(PCH - 1, 0)
      if with_deg:
        fire_deg(PCH - 1, 0)
      wait_scatter(1)
      if with_deg:
        wait_deg(1)

      wait_scatter(0)
      if with_deg:
        wait_deg(0)

    plsc.subcore_barrier()

    @pl.when(s < N // DEG_CHUNK)
    def _():
      pltpu.sync_copy(acc_sh.at[pl.ds(s * DEG_CHUNK, DEG_CHUNK)],
                      agg_out.at[c, pl.ds(s * DEG_CHUNK, DEG_CHUNK)])
    if with_deg:
      @pl.when(s < N // DEG_CHUNK)
      def _():
        pltpu.sync_copy(deg_sh.at[pl.ds(s * DEG_CHUNK, DEG_CHUNK)],
                        degbuf.at[pl.ds(0, DEG_CHUNK)])
        pltpu.sync_copy(degbuf.at[pl.ds(0, DEG_CHUNK)],
                        deg_out.at[pl.ds(c * N + s * DEG_CHUNK, DEG_CHUNK)])

  return pl.kernel(body, out_type=out_type, mesh=mesh,
                   scratch_types=scratch)


_sc_agg_deg = _make_sc_agg(D_IN, with_deg=True)
_sc_agg_p = _make_sc_agg(CP, with_deg=False)

R = 1000


def _tc1_body(a0, a1, d0, d1, x, w1, b1, w2, p_out):
  num = a0[0] + a1[0] + x[...]
  den = d0[0] + d1[0] + 1.0
  h = num / den
  h = jnp.maximum(jnp.dot(h, w1[...], preferred_element_type=jnp.float32)
                  + b1[...], 0.0)
  p_out[...] = jnp.dot(h, w2[...], preferred_element_type=jnp.float32)


def _tc2_body(g0, g1, d0, d1, p, b2, out):
  den = d0[0] + d1[0] + 1.0
  t = (g0[0] + g1[0] + p[...]) / den + b2[...]
  out[...] = t[:, :C]


def kernel(x, edge_index, W1, b1, W2, b2):
  src3 = edge_index[0].astype(jnp.int32).reshape(NW * PH, PCH, K)
  dst3 = edge_index[1].astype(jnp.int32).reshape(NW * PH, PCH, K)
  z2 = jnp.zeros((N, D_IN), jnp.float32)
  w2p = jnp.pad(W2, ((0, 0), (0, CP - C)))
  b2p = jnp.pad(b2, (0, CP - C)).reshape(1, CP)

  aggp, degp = _sc_agg_deg(x, src3, dst3, z2)
  degp3 = degp.reshape(NC, N, 1)

  grid = (N // R,)
  p = pl.pallas_call(
      _tc1_body,
      grid=grid,
      in_specs=[
          pl.BlockSpec((1, R, D_IN), lambda i: (0, i, 0)),
          pl.BlockSpec((1, R, D_IN), lambda i: (1, i, 0)),
          pl.BlockSpec((1, R, 1), lambda i: (0, i, 0)),
          pl.BlockSpec((1, R, 1), lambda i: (1, i, 0)),
          pl.BlockSpec((R, D_IN), lambda i: (i, 0)),
          pl.BlockSpec((D_IN, D_HID), lambda i: (0, 0)),
          pl.BlockSpec((1, D_HID), lambda i: (0, 0)),
          pl.BlockSpec((D_HID, CP), lambda i: (0, 0)),
      ],
      out_specs=pl.BlockSpec((R, CP), lambda i: (i, 0)),
      out_shape=jax.ShapeDtypeStruct((N, CP), jnp.float32),
  )(aggp, aggp, degp3, degp3, x, W1, b1.reshape(1, D_HID), w2p)

  (gp,) = _sc_agg_p(p, src3, dst3, z2)

  out = pl.pallas_call(
      _tc2_body,
      grid=grid,
      in_specs=[
          pl.BlockSpec((1, R, CP), lambda i: (0, i, 0)),
          pl.BlockSpec((1, R, CP), lambda i: (1, i, 0)),
          pl.BlockSpec((1, R, 1), lambda i: (0, i, 0)),
          pl.BlockSpec((1, R, 1), lambda i: (1, i, 0)),
          pl.BlockSpec((R, CP), lambda i: (i, 0)),
          pl.BlockSpec((1, CP), lambda i: (0, 0)),
      ],
      out_specs=pl.BlockSpec((R, C), lambda i: (i, 0)),
      out_shape=jax.ShapeDtypeStruct((N, C), jnp.float32),
  )(gp, gp, degp3, degp3, p, b2p)

  return out


# speedup vs baseline: 2.7395x; 1.2050x over previous
import functools

import jax
import jax.numpy as jnp
from jax import lax
from jax.experimental import pallas as pl
from jax.experimental.pallas import tpu as pltpu
from jax.experimental.pallas import tpu_sc as plsc

N = 10000
E = 320000
D_IN = 128
D_HID = 128
C = 40
CP = 128

NC, NS = 2, 16
NW = NC * NS
E_W = E // NW
K = 80
NB = 2
PH = 5
PCH = E_W // (PH * K)
NP = N
DEG_CHUNK = 1000


def _make_sc_agg(D, with_deg):
  mesh = plsc.VectorSubcoreMesh(
      core_axis_name="c", subcore_axis_name="s",
      num_cores=NC, num_subcores=NS)

  out_type = [jax.ShapeDtypeStruct((NC, N, D), jnp.float32)]
  scratch = [
      pltpu.VMEM((PCH, K), jnp.int32),
      pltpu.VMEM((PCH, K), jnp.int32),
  ] + [pltpu.VMEM((K, D), jnp.float32) for _ in range(NB)] + [
      pltpu.VMEM_SHARED((NP, D), jnp.float32),
  ] + [pltpu.SemaphoreType.DMA for _ in range(2 * NB)]
  if with_deg:
    out_type.append(jax.ShapeDtypeStruct((NC * N,), jnp.float32))
    scratch += [
        pltpu.VMEM((K,), jnp.float32),
        pltpu.VMEM_SHARED((NP,), jnp.float32),
        pltpu.VMEM((1008,), jnp.float32),
    ] + [pltpu.SemaphoreType.DMA for _ in range(NB)]

  def body(*refs):
    x_hbm, src_hbm, dst_hbm, z2_hbm = refs[:4]
    nout = 2 if with_deg else 1
    agg_out = refs[4]
    k = 4 + nout
    srcv, dstv = refs[k], refs[k + 1]
    rows = refs[k + 2:k + 2 + NB]
    acc_sh = refs[k + 2 + NB]
    gsem = refs[k + 3 + NB:k + 3 + 2 * NB]
    ssem = refs[k + 3 + 2 * NB:k + 3 + 3 * NB]
    if with_deg:
      deg_out = refs[5]
      onesv, deg_sh, degbuf = refs[k + 3 + 3 * NB:k + 6 + 3 * NB]
      dsem = refs[k + 6 + 3 * NB:k + 6 + 4 * NB]

    c = lax.axis_index("c")
    s = lax.axis_index("s")
    wid = c * NS + s

    @pl.when(s < N // DEG_CHUNK)
    def _():
      pltpu.sync_copy(z2_hbm.at[pl.ds(s * DEG_CHUNK, DEG_CHUNK)],
                      acc_sh.at[pl.ds(s * DEG_CHUNK, DEG_CHUNK)])
    if with_deg:
      for i in range(1008 // 16):
        degbuf[pl.ds(i * 16, 16)] = jnp.zeros((16,), jnp.float32)
      @pl.when(s < N // DEG_CHUNK)
      def _():
        pltpu.sync_copy(degbuf.at[pl.ds(0, DEG_CHUNK)],
                        deg_sh.at[pl.ds(s * DEG_CHUNK, DEG_CHUNK)])
      for i in range(K // 16):
        onesv[pl.ds(i * 16, 16)] = jnp.full((16,), 1.0, jnp.float32)

    plsc.subcore_barrier()

    def fire_gather(jj, b):
      pltpu.async_copy(x_hbm.at[srcv.at[jj]], rows[b], gsem[b])

    def wait_gather(b):
      pltpu.make_async_copy(x_hbm.at[srcv.at[0]], rows[b], gsem[b]).wait()

    def fire_scatter(jj, b):
      pltpu.async_copy(rows[b], acc_sh.at[dstv.at[jj]], ssem[b], add=True)

    def wait_scatter(b):
      pltpu.make_async_copy(rows[b], acc_sh.at[dstv.at[0]], ssem[b]).wait()

    if with_deg:
      def fire_deg(jj):
        pltpu.async_copy(onesv, deg_sh.at[dstv.at[jj]], dsem[0], add=True)

      def drain_deg():
        def dwait(i, carry):
          pltpu.make_async_copy(onesv, deg_sh.at[dstv.at[0]],
                                dsem[0]).wait()
          return carry
        lax.fori_loop(0, PCH, dwait, 0)

    for p in range(PH):
      pltpu.sync_copy(src_hbm.at[wid * PH + p], srcv)
      pltpu.sync_copy(dst_hbm.at[wid * PH + p], dstv)

      fire_gather(0, 0)

      def tbody(t, carry):
        for u in range(2):
          i = 2 * t + u
          b = u
          bn = 1 - u
          if u == 0:
            @pl.when(i >= 1)
            def _():
              wait_scatter(bn)
          else:
            wait_scatter(bn)
          fire_gather(i + 1, bn)
          wait_gather(b)
          fire_scatter(i, b)
          if with_deg:
            fire_deg(i)
        return carry

      lax.fori_loop(0, PCH // 2, tbody, 0)

      wait_scatter(1)
      wait_gather(0)
      fire_scatter(PCH - 1, 0)
      if with_deg:
        fire_deg(PCH - 1)

      wait_scatter(0)
      if with_deg:
        drain_deg()

    plsc.subcore_barrier()

    @pl.when(s < N // DEG_CHUNK)
    def _():
      pltpu.sync_copy(acc_sh.at[pl.ds(s * DEG_CHUNK, DEG_CHUNK)],
                      agg_out.at[c, pl.ds(s * DEG_CHUNK, DEG_CHUNK)])
    if with_deg:
      @pl.when(s < N // DEG_CHUNK)
      def _():
        pltpu.sync_copy(deg_sh.at[pl.ds(s * DEG_CHUNK, DEG_CHUNK)],
                        degbuf.at[pl.ds(0, DEG_CHUNK)])
        pltpu.sync_copy(degbuf.at[pl.ds(0, DEG_CHUNK)],
                        deg_out.at[pl.ds(c * N + s * DEG_CHUNK, DEG_CHUNK)])

  return pl.kernel(body, out_type=out_type, mesh=mesh,
                   scratch_types=scratch)


_sc_agg_deg = _make_sc_agg(D_IN, with_deg=True)
_sc_agg_p = _make_sc_agg(CP, with_deg=False)

R = 1000


def _tc1_body(a0, a1, d0, d1, x, w1, b1, w2, p_out):
  num = a0[0] + a1[0] + x[...]
  den = d0[0] + d1[0] + 1.0
  h = num / den
  h = jnp.maximum(jnp.dot(h, w1[...], preferred_element_type=jnp.float32)
                  + b1[...], 0.0)
  p_out[...] = jnp.dot(h, w2[...], preferred_element_type=jnp.float32)


def _tc2_body(g0, g1, d0, d1, p, b2, out):
  den = d0[0] + d1[0] + 1.0
  t = (g0[0] + g1[0] + p[...]) / den + b2[...]
  out[...] = t[:, :C]


def kernel(x, edge_index, W1, b1, W2, b2):
  src3 = edge_index[0].astype(jnp.int32).reshape(NW * PH, PCH, K)
  dst3 = edge_index[1].astype(jnp.int32).reshape(NW * PH, PCH, K)
  z2 = jnp.zeros((N, D_IN), jnp.float32)
  w2p = jnp.pad(W2, ((0, 0), (0, CP - C)))
  b2p = jnp.pad(b2, (0, CP - C)).reshape(1, CP)

  aggp, degp = _sc_agg_deg(x, src3, dst3, z2)
  degp3 = degp.reshape(NC, N, 1)

  grid = (N // R,)
  p = pl.pallas_call(
      _tc1_body,
      grid=grid,
      in_specs=[
          pl.BlockSpec((1, R, D_IN), lambda i: (0, i, 0)),
          pl.BlockSpec((1, R, D_IN), lambda i: (1, i, 0)),
          pl.BlockSpec((1, R, 1), lambda i: (0, i, 0)),
          pl.BlockSpec((1, R, 1), lambda i: (1, i, 0)),
          pl.BlockSpec((R, D_IN), lambda i: (i, 0)),
          pl.BlockSpec((D_IN, D_HID), lambda i: (0, 0)),
          pl.BlockSpec((1, D_HID), lambda i: (0, 0)),
          pl.BlockSpec((D_HID, CP), lambda i: (0, 0)),
      ],
      out_specs=pl.BlockSpec((R, CP), lambda i: (i, 0)),
      out_shape=jax.ShapeDtypeStruct((N, CP), jnp.float32),
  )(aggp, aggp, degp3, degp3, x, W1, b1.reshape(1, D_HID), w2p)

  (gp,) = _sc_agg_p(p, src3, dst3, z2)

  out = pl.pallas_call(
      _tc2_body,
      grid=grid,
      in_specs=[
          pl.BlockSpec((1, R, CP), lambda i: (0, i, 0)),
          pl.BlockSpec((1, R, CP), lambda i: (1, i, 0)),
          pl.BlockSpec((1, R, 1), lambda i: (0, i, 0)),
          pl.BlockSpec((1, R, 1), lambda i: (1, i, 0)),
          pl.BlockSpec((R, CP), lambda i: (i, 0)),
          pl.BlockSpec((1, CP), lambda i: (0, 0)),
      ],
      out_specs=pl.BlockSpec((R, C), lambda i: (i, 0)),
      out_shape=jax.ShapeDtypeStruct((N, C), jnp.float32),
  )(gp, gp, degp3, degp3, p, b2p)

  return out


# speedup vs baseline: 3.0758x; 1.1227x over previous
import functools

import jax
import jax.numpy as jnp
from jax import lax
from jax.experimental import pallas as pl
from jax.experimental.pallas import tpu as pltpu
from jax.experimental.pallas import tpu_sc as plsc

N = 10000
E = 320000
D_IN = 128
D_HID = 128
C = 40
CP = 128

NC, NS = 2, 16
NW = NC * NS
E_W = E // NW
K = 80
NB = 3
PH = 5
PCH = E_W // (PH * K)
NP = N
DEG_CHUNK = 1000


def _make_sc_agg(D, with_deg):
  mesh = plsc.VectorSubcoreMesh(
      core_axis_name="c", subcore_axis_name="s",
      num_cores=NC, num_subcores=NS)

  out_type = [jax.ShapeDtypeStruct((NC, N, D), jnp.float32)]
  scratch = [
      pltpu.VMEM((PCH, K), jnp.int32),
      pltpu.VMEM((PCH, K), jnp.int32),
  ] + [pltpu.VMEM((K, D), jnp.float32) for _ in range(NB)] + [
      pltpu.VMEM_SHARED((NP, D), jnp.float32),
  ] + [pltpu.SemaphoreType.DMA for _ in range(2 * NB)]
  if with_deg:
    out_type.append(jax.ShapeDtypeStruct((NC * N,), jnp.float32))
    scratch += [
        pltpu.VMEM((K,), jnp.float32),
        pltpu.VMEM_SHARED((NP,), jnp.float32),
        pltpu.VMEM((1008,), jnp.float32),
    ] + [pltpu.SemaphoreType.DMA for _ in range(NB)]

  def body(*refs):
    x_hbm, src_hbm, dst_hbm, z2_hbm = refs[:4]
    nout = 2 if with_deg else 1
    agg_out = refs[4]
    k = 4 + nout
    srcv, dstv = refs[k], refs[k + 1]
    rows = refs[k + 2:k + 2 + NB]
    acc_sh = refs[k + 2 + NB]
    gsem = refs[k + 3 + NB:k + 3 + 2 * NB]
    ssem = refs[k + 3 + 2 * NB:k + 3 + 3 * NB]
    if with_deg:
      deg_out = refs[5]
      onesv, deg_sh, degbuf = refs[k + 3 + 3 * NB:k + 6 + 3 * NB]
      dsem = refs[k + 6 + 3 * NB:k + 6 + 4 * NB]

    c = lax.axis_index("c")
    s = lax.axis_index("s")
    wid = c * NS + s

    @pl.when(s < N // DEG_CHUNK)
    def _():
      pltpu.sync_copy(z2_hbm.at[pl.ds(s * DEG_CHUNK, DEG_CHUNK)],
                      acc_sh.at[pl.ds(s * DEG_CHUNK, DEG_CHUNK)])
    if with_deg:
      for i in range(1008 // 16):
        degbuf[pl.ds(i * 16, 16)] = jnp.zeros((16,), jnp.float32)
      @pl.when(s < N // DEG_CHUNK)
      def _():
        pltpu.sync_copy(degbuf.at[pl.ds(0, DEG_CHUNK)],
                        deg_sh.at[pl.ds(s * DEG_CHUNK, DEG_CHUNK)])
      for i in range(K // 16):
        onesv[pl.ds(i * 16, 16)] = jnp.full((16,), 1.0, jnp.float32)

    plsc.subcore_barrier()

    def fire_gather(jj, b):
      pltpu.async_copy(x_hbm.at[srcv.at[jj]], rows[b], gsem[b])

    def wait_gather(b):
      pltpu.make_async_copy(x_hbm.at[srcv.at[0]], rows[b], gsem[b]).wait()

    def fire_scatter(jj, b):
      pltpu.async_copy(rows[b], acc_sh.at[dstv.at[jj]], ssem[b], add=True)

    def wait_scatter(b):
      pltpu.make_async_copy(rows[b], acc_sh.at[dstv.at[0]], ssem[b]).wait()

    if with_deg:
      def fire_deg(jj):
        pltpu.async_copy(onesv, deg_sh.at[dstv.at[jj]], dsem[0], add=True)

      def drain_deg():
        def dwait(i, carry):
          pltpu.make_async_copy(onesv, deg_sh.at[dstv.at[0]],
                                dsem[0]).wait()
          return carry
        lax.fori_loop(0, PCH, dwait, 0)

    for p in range(PH):
      pltpu.sync_copy(src_hbm.at[wid * PH + p], srcv)
      pltpu.sync_copy(dst_hbm.at[wid * PH + p], dstv)

      fire_gather(0, 0)
      fire_gather(1, 1)

      def tbody(t, carry):
        for u in range(3):
          i = 3 * t + u
          b = u
          wait_gather(b)
          if u == 0:
            @pl.when(i >= 1)
            def _():
              wait_scatter(2)
          else:
            wait_scatter(u - 1)
          fire_scatter(i, b)
          if with_deg:
            fire_deg(i)
          if u < 2:
            fire_gather(i + 2, (u + 2) % 3)
          else:
            @pl.when(i + 2 < PCH)
            def _():
              fire_gather(i + 2, 1)
        return carry

      lax.fori_loop(0, PCH // 3, tbody, 0)

      wait_gather(0)
      wait_scatter(2)
      fire_scatter(PCH - 1, 0)
      if with_deg:
        fire_deg(PCH - 1)

      wait_scatter(0)
      if with_deg:
        drain_deg()

    plsc.subcore_barrier()

    @pl.when(s < N // DEG_CHUNK)
    def _():
      pltpu.sync_copy(acc_sh.at[pl.ds(s * DEG_CHUNK, DEG_CHUNK)],
                      agg_out.at[c, pl.ds(s * DEG_CHUNK, DEG_CHUNK)])
    if with_deg:
      @pl.when(s < N // DEG_CHUNK)
      def _():
        pltpu.sync_copy(deg_sh.at[pl.ds(s * DEG_CHUNK, DEG_CHUNK)],
                        degbuf.at[pl.ds(0, DEG_CHUNK)])
        pltpu.sync_copy(degbuf.at[pl.ds(0, DEG_CHUNK)],
                        deg_out.at[pl.ds(c * N + s * DEG_CHUNK, DEG_CHUNK)])

  return pl.kernel(body, out_type=out_type, mesh=mesh,
                   scratch_types=scratch)


_sc_agg_deg = _make_sc_agg(D_IN, with_deg=True)
_sc_agg_p = _make_sc_agg(CP, with_deg=False)

R = 1000


def _tc1_body(a0, a1, d0, d1, x, w1, b1, w2, p_out):
  num = a0[0] + a1[0] + x[...]
  den = d0[0] + d1[0] + 1.0
  h = num / den
  h = jnp.maximum(jnp.dot(h, w1[...], preferred_element_type=jnp.float32)
                  + b1[...], 0.0)
  p_out[...] = jnp.dot(h, w2[...], preferred_element_type=jnp.float32)


def _tc2_body(g0, g1, d0, d1, p, b2, out):
  den = d0[0] + d1[0] + 1.0
  t = (g0[0] + g1[0] + p[...]) / den + b2[...]
  out[...] = t[:, :C]


def kernel(x, edge_index, W1, b1, W2, b2):
  src3 = edge_index[0].astype(jnp.int32).reshape(NW * PH, PCH, K)
  dst3 = edge_index[1].astype(jnp.int32).reshape(NW * PH, PCH, K)
  z2 = jnp.zeros((N, D_IN), jnp.float32)
  w2p = jnp.pad(W2, ((0, 0), (0, CP - C)))
  b2p = jnp.pad(b2, (0, CP - C)).reshape(1, CP)

  aggp, degp = _sc_agg_deg(x, src3, dst3, z2)
  degp3 = degp.reshape(NC, N, 1)

  grid = (N // R,)
  p = pl.pallas_call(
      _tc1_body,
      grid=grid,
      in_specs=[
          pl.BlockSpec((1, R, D_IN), lambda i: (0, i, 0)),
          pl.BlockSpec((1, R, D_IN), lambda i: (1, i, 0)),
          pl.BlockSpec((1, R, 1), lambda i: (0, i, 0)),
          pl.BlockSpec((1, R, 1), lambda i: (1, i, 0)),
          pl.BlockSpec((R, D_IN), lambda i: (i, 0)),
          pl.BlockSpec((D_IN, D_HID), lambda i: (0, 0)),
          pl.BlockSpec((1, D_HID), lambda i: (0, 0)),
          pl.BlockSpec((D_HID, CP), lambda i: (0, 0)),
      ],
      out_specs=pl.BlockSpec((R, CP), lambda i: (i, 0)),
      out_shape=jax.ShapeDtypeStruct((N, CP), jnp.float32),
  )(aggp, aggp, degp3, degp3, x, W1, b1.reshape(1, D_HID), w2p)

  (gp,) = _sc_agg_p(p, src3, dst3, z2)

  out = pl.pallas_call(
      _tc2_body,
      grid=grid,
      in_specs=[
          pl.BlockSpec((1, R, CP), lambda i: (0, i, 0)),
          pl.BlockSpec((1, R, CP), lambda i: (1, i, 0)),
          pl.BlockSpec((1, R, 1), lambda i: (0, i, 0)),
          pl.BlockSpec((1, R, 1), lambda i: (1, i, 0)),
          pl.BlockSpec((R, CP), lambda i: (i, 0)),
          pl.BlockSpec((1, CP), lambda i: (0, 0)),
      ],
      out_specs=pl.BlockSpec((R, C), lambda i: (i, 0)),
      out_shape=jax.ShapeDtypeStruct((N, C), jnp.float32),
  )(gp, gp, degp3, degp3, p, b2p)

  return out


# speedup vs baseline: 3.1766x; 1.0328x over previous
import functools

import jax
import jax.numpy as jnp
from jax import lax
from jax.experimental import pallas as pl
from jax.experimental.pallas import tpu as pltpu
from jax.experimental.pallas import tpu_sc as plsc

N = 10000
E = 320000
D_IN = 128
D_HID = 128
C = 40
CP = 128

NC, NS = 2, 16
NW = NC * NS
E_W = E // NW
K = 80
NB = 3
PH = 5
PCH = E_W // (PH * K)
NP = N
DEG_CHUNK = 1000


def _make_sc_agg(D, with_deg):
  mesh = plsc.VectorSubcoreMesh(
      core_axis_name="c", subcore_axis_name="s",
      num_cores=NC, num_subcores=NS)

  out_type = [jax.ShapeDtypeStruct((NC, N, D), jnp.float32)]
  scratch = [
      pltpu.VMEM((PCH, K), jnp.int32),
      pltpu.VMEM((PCH, K), jnp.int32),
      pltpu.VMEM((PCH, K), jnp.int32),
      pltpu.VMEM((PCH, K), jnp.int32),
      pltpu.SemaphoreType.DMA,
      pltpu.SemaphoreType.DMA,
  ] + [pltpu.VMEM((K, D), jnp.float32) for _ in range(NB)] + [
      pltpu.VMEM_SHARED((NP, D), jnp.float32),
  ] + [pltpu.SemaphoreType.DMA for _ in range(2 * NB)]
  if with_deg:
    out_type.append(jax.ShapeDtypeStruct((NC * N,), jnp.float32))
    scratch += [
        pltpu.VMEM((K,), jnp.float32),
        pltpu.VMEM_SHARED((NP,), jnp.float32),
        pltpu.VMEM((1008,), jnp.float32),
    ] + [pltpu.SemaphoreType.DMA for _ in range(NB)]

  def body(*refs):
    x_hbm, src_hbm, dst_hbm, z2_hbm = refs[:4]
    nout = 2 if with_deg else 1
    agg_out = refs[4]
    k = 4 + nout
    srcvs = [refs[k], refs[k + 2]]
    dstvs = [refs[k + 1], refs[k + 3]]
    isems = [refs[k + 4], refs[k + 5]]
    k = k + 6
    rows = refs[k:k + NB]
    acc_sh = refs[k + NB]
    gsem = refs[k + 1 + NB:k + 1 + 2 * NB]
    ssem = refs[k + 1 + 2 * NB:k + 1 + 3 * NB]
    if with_deg:
      deg_out = refs[5]
      onesv, deg_sh, degbuf = refs[k + 1 + 3 * NB:k + 4 + 3 * NB]
      dsem = refs[k + 4 + 3 * NB:k + 4 + 4 * NB]

    c = lax.axis_index("c")
    s = lax.axis_index("s")
    wid = c * NS + s

    @pl.when(s < N // DEG_CHUNK)
    def _():
      pltpu.sync_copy(z2_hbm.at[pl.ds(s * DEG_CHUNK, DEG_CHUNK)],
                      acc_sh.at[pl.ds(s * DEG_CHUNK, DEG_CHUNK)])
    if with_deg:
      for i in range(1008 // 16):
        degbuf[pl.ds(i * 16, 16)] = jnp.zeros((16,), jnp.float32)
      @pl.when(s < N // DEG_CHUNK)
      def _():
        pltpu.sync_copy(degbuf.at[pl.ds(0, DEG_CHUNK)],
                        deg_sh.at[pl.ds(s * DEG_CHUNK, DEG_CHUNK)])
      for i in range(K // 16):
        onesv[pl.ds(i * 16, 16)] = jnp.full((16,), 1.0, jnp.float32)

    plsc.subcore_barrier()

    def fire_idx(p, pb):
      pltpu.async_copy(src_hbm.at[wid * PH + p], srcvs[pb], isems[pb])
      pltpu.async_copy(dst_hbm.at[wid * PH + p], dstvs[pb], isems[pb])

    def wait_idx(pb):
      pltpu.make_async_copy(src_hbm.at[wid * PH], srcvs[pb], isems[pb]).wait()
      pltpu.make_async_copy(dst_hbm.at[wid * PH], dstvs[pb], isems[pb]).wait()

    fire_idx(0, 0)

    for p in range(PH):
      pb = p % 2
      srcv = srcvs[pb]
      dstv = dstvs[pb]

      def fire_gather(jj, b):
        pltpu.async_copy(x_hbm.at[srcv.at[jj]], rows[b], gsem[b])

      def wait_gather(b):
        pltpu.make_async_copy(x_hbm.at[srcv.at[0]], rows[b], gsem[b]).wait()

      def fire_scatter(jj, b):
        pltpu.async_copy(rows[b], acc_sh.at[dstv.at[jj]], ssem[b], add=True)

      def wait_scatter(b):
        pltpu.make_async_copy(rows[b], acc_sh.at[dstv.at[0]],
                              ssem[b]).wait()

      if with_deg:
        def fire_deg(jj):
          pltpu.async_copy(onesv, deg_sh.at[dstv.at[jj]], dsem[0], add=True)

        def drain_deg():
          def dwait(i, carry):
            pltpu.make_async_copy(onesv, deg_sh.at[dstv.at[0]],
                                  dsem[0]).wait()
            return carry
          lax.fori_loop(0, PCH, dwait, 0)

      wait_idx(pb)
      if p + 1 < PH:
        fire_idx(p + 1, 1 - pb)

      fire_gather(0, 0)
      fire_gather(1, 1)

      def tbody(t, carry):
        for u in range(3):
          i = 3 * t + u
          b = u
          wait_gather(b)
          if u == 0:
            @pl.when(i >= 1)
            def _():
              wait_scatter(2)
          else:
            wait_scatter(u - 1)
          fire_scatter(i, b)
          if with_deg:
            fire_deg(i)
          if u < 2:
            fire_gather(i + 2, (u + 2) % 3)
          else:
            @pl.when(i + 2 < PCH)
            def _():
              fire_gather(i + 2, 1)
        return carry

      lax.fori_loop(0, PCH // 3, tbody, 0)

      wait_gather(0)
      wait_scatter(2)
      fire_scatter(PCH - 1, 0)
      if with_deg:
        fire_deg(PCH - 1)

      wait_scatter(0)
      if with_deg:
        drain_deg()

    plsc.subcore_barrier()

    @pl.when(s < N // DEG_CHUNK)
    def _():
      pltpu.sync_copy(acc_sh.at[pl.ds(s * DEG_CHUNK, DEG_CHUNK)],
                      agg_out.at[c, pl.ds(s * DEG_CHUNK, DEG_CHUNK)])
    if with_deg:
      @pl.when(s < N // DEG_CHUNK)
      def _():
        pltpu.sync_copy(deg_sh.at[pl.ds(s * DEG_CHUNK, DEG_CHUNK)],
                        degbuf.at[pl.ds(0, DEG_CHUNK)])
        pltpu.sync_copy(degbuf.at[pl.ds(0, DEG_CHUNK)],
                        deg_out.at[pl.ds(c * N + s * DEG_CHUNK, DEG_CHUNK)])

  return pl.kernel(body, out_type=out_type, mesh=mesh,
                   scratch_types=scratch)


_sc_agg_deg = _make_sc_agg(D_IN, with_deg=True)
_sc_agg_p = _make_sc_agg(CP, with_deg=False)

R = 1000


def _tc1_body(a0, a1, d0, d1, x, w1, b1, w2, p_out):
  num = a0[0] + a1[0] + x[...]
  den = d0[0] + d1[0] + 1.0
  h = num / den
  h = jnp.maximum(jnp.dot(h, w1[...], preferred_element_type=jnp.float32)
                  + b1[...], 0.0)
  p_out[...] = jnp.dot(h, w2[...], preferred_element_type=jnp.float32)


def _tc2_body(g0, g1, d0, d1, p, b2, out):
  den = d0[0] + d1[0] + 1.0
  t = (g0[0] + g1[0] + p[...]) / den + b2[...]
  out[...] = t[:, :C]


def kernel(x, edge_index, W1, b1, W2, b2):
  src3 = edge_index[0].astype(jnp.int32).reshape(NW * PH, PCH, K)
  dst3 = edge_index[1].astype(jnp.int32).reshape(NW * PH, PCH, K)
  z2 = jnp.zeros((N, D_IN), jnp.float32)
  w2p = jnp.pad(W2, ((0, 0), (0, CP - C)))
  b2p = jnp.pad(b2, (0, CP - C)).reshape(1, CP)

  aggp, degp = _sc_agg_deg(x, src3, dst3, z2)
  degp3 = degp.reshape(NC, N, 1)

  grid = (N // R,)
  p = pl.pallas_call(
      _tc1_body,
      grid=grid,
      in_specs=[
          pl.BlockSpec((1, R, D_IN), lambda i: (0, i, 0)),
          pl.BlockSpec((1, R, D_IN), lambda i: (1, i, 0)),
          pl.BlockSpec((1, R, 1), lambda i: (0, i, 0)),
          pl.BlockSpec((1, R, 1), lambda i: (1, i, 0)),
          pl.BlockSpec((R, D_IN), lambda i: (i, 0)),
          pl.BlockSpec((D_IN, D_HID), lambda i: (0, 0)),
          pl.BlockSpec((1, D_HID), lambda i: (0, 0)),
          pl.BlockSpec((D_HID, CP), lambda i: (0, 0)),
      ],
      out_specs=pl.BlockSpec((R, CP), lambda i: (i, 0)),
      out_shape=jax.ShapeDtypeStruct((N, CP), jnp.float32),
  )(aggp, aggp, degp3, degp3, x, W1, b1.reshape(1, D_HID), w2p)

  (gp,) = _sc_agg_p(p, src3, dst3, z2)

  out = pl.pallas_call(
      _tc2_body,
      grid=grid,
      in_specs=[
          pl.BlockSpec((1, R, CP), lambda i: (0, i, 0)),
          pl.BlockSpec((1, R, CP), lambda i: (1, i, 0)),
          pl.BlockSpec((1, R, 1), lambda i: (0, i, 0)),
          pl.BlockSpec((1, R, 1), lambda i: (1, i, 0)),
          pl.BlockSpec((R, CP), lambda i: (i, 0)),
          pl.BlockSpec((1, CP), lambda i: (0, 0)),
      ],
      out_specs=pl.BlockSpec((R, C), lambda i: (i, 0)),
      out_shape=jax.ShapeDtypeStruct((N, C), jnp.float32),
  )(gp, gp, degp3, degp3, p, b2p)

  return out


# speedup vs baseline: 3.1954x; 1.0059x over previous
import functools

import jax
import jax.numpy as jnp
from jax import lax
from jax.experimental import pallas as pl
from jax.experimental.pallas import tpu as pltpu
from jax.experimental.pallas import tpu_sc as plsc

N = 10000
E = 320000
D_IN = 128
D_HID = 128
C = 40
CP = 128

NC, NS = 2, 16
NW = NC * NS
E_W = E // NW
K = 80
NB = 3
PH = 5
PCH = E_W // (PH * K)
NP = N
DEG_CHUNK = 1000


def _make_sc_agg(D, with_deg):
  mesh = plsc.VectorSubcoreMesh(
      core_axis_name="c", subcore_axis_name="s",
      num_cores=NC, num_subcores=NS)

  out_type = [jax.ShapeDtypeStruct((NC, N, D), jnp.float32)]
  scratch = [
      pltpu.VMEM((PCH, K), jnp.int32),
      pltpu.VMEM((PCH, K), jnp.int32),
      pltpu.VMEM((PCH, K), jnp.int32),
      pltpu.VMEM((PCH, K), jnp.int32),
      pltpu.SemaphoreType.DMA,
      pltpu.SemaphoreType.DMA,
  ] + [pltpu.VMEM((K, D), jnp.float32) for _ in range(NB)] + [
      pltpu.VMEM_SHARED((NP, D), jnp.float32),
  ] + [pltpu.SemaphoreType.DMA for _ in range(2 * NB)]
  if with_deg:
    out_type.append(jax.ShapeDtypeStruct((NC * N,), jnp.float32))
    scratch += [
        pltpu.VMEM((K,), jnp.float32),
        pltpu.VMEM_SHARED((NP,), jnp.float32),
        pltpu.VMEM((1008,), jnp.float32),
    ] + [pltpu.SemaphoreType.DMA for _ in range(NB)]

  def body(*refs):
    x_hbm, src_hbm, dst_hbm, z2_hbm = refs[:4]
    nout = 2 if with_deg else 1
    agg_out = refs[4]
    k = 4 + nout
    srcvs = [refs[k], refs[k + 2]]
    dstvs = [refs[k + 1], refs[k + 3]]
    isems = [refs[k + 4], refs[k + 5]]
    k = k + 6
    rows = refs[k:k + NB]
    acc_sh = refs[k + NB]
    gsem = refs[k + 1 + NB:k + 1 + 2 * NB]
    ssem = refs[k + 1 + 2 * NB:k + 1 + 3 * NB]
    if with_deg:
      deg_out = refs[5]
      onesv, deg_sh, degbuf = refs[k + 1 + 3 * NB:k + 4 + 3 * NB]
      dsem = refs[k + 4 + 3 * NB:k + 4 + 4 * NB]

    c = lax.axis_index("c")
    s = lax.axis_index("s")
    wid = c * NS + s

    def fire_idx(p, pb):
      pltpu.async_copy(src_hbm.at[wid * PH + p], srcvs[pb], isems[pb])
      pltpu.async_copy(dst_hbm.at[wid * PH + p], dstvs[pb], isems[pb])

    def wait_idx(pb):
      pltpu.make_async_copy(src_hbm.at[wid * PH], srcvs[pb], isems[pb]).wait()
      pltpu.make_async_copy(dst_hbm.at[wid * PH], dstvs[pb], isems[pb]).wait()

    fire_idx(0, 0)
    fire_idx(1, 1)

    @pl.when(s < NS - 1)
    def _():
      pltpu.sync_copy(z2_hbm.at[pl.ds(s * 624, 624)],
                      acc_sh.at[pl.ds(s * 624, 624)])
    @pl.when(s == NS - 1)
    def _():
      pltpu.sync_copy(z2_hbm.at[pl.ds(9360, 640)],
                      acc_sh.at[pl.ds(9360, 640)])
    if with_deg:
      for i in range(1008 // 16):
        degbuf[pl.ds(i * 16, 16)] = jnp.zeros((16,), jnp.float32)
      @pl.when(s < NS - 1)
      def _():
        pltpu.sync_copy(degbuf.at[pl.ds(0, 624)],
                        deg_sh.at[pl.ds(s * 624, 624)])
      @pl.when(s == NS - 1)
      def _():
        pltpu.sync_copy(degbuf.at[pl.ds(0, 640)],
                        deg_sh.at[pl.ds(9360, 640)])
      for i in range(K // 16):
        onesv[pl.ds(i * 16, 16)] = jnp.full((16,), 1.0, jnp.float32)

    plsc.subcore_barrier()

    for p in range(PH):
      pb = p % 2
      srcv = srcvs[pb]
      dstv = dstvs[pb]

      def fire_gather(jj, b):
        pltpu.async_copy(x_hbm.at[srcv.at[jj]], rows[b], gsem[b])

      def wait_gather(b):
        pltpu.make_async_copy(x_hbm.at[srcv.at[0]], rows[b], gsem[b]).wait()

      def fire_scatter(jj, b):
        pltpu.async_copy(rows[b], acc_sh.at[dstv.at[jj]], ssem[b], add=True)

      def wait_scatter(b):
        pltpu.make_async_copy(rows[b], acc_sh.at[dstv.at[0]],
                              ssem[b]).wait()

      if with_deg:
        def fire_deg(jj):
          pltpu.async_copy(onesv, deg_sh.at[dstv.at[jj]], dsem[0], add=True)

        def drain_deg():
          def dwait(i, carry):
            pltpu.make_async_copy(onesv, deg_sh.at[dstv.at[0]],
                                  dsem[0]).wait()
            return carry
          lax.fori_loop(0, PCH, dwait, 0)

      wait_idx(pb)
      if 1 <= p < PH - 1:
        fire_idx(p + 1, 1 - pb)

      fire_gather(0, 0)
      fire_gather(1, 1)

      def tbody(t, carry):
        for u in range(3):
          i = 3 * t + u
          b = u
          wait_gather(b)
          if u == 0:
            @pl.when(i >= 1)
            def _():
              wait_scatter(2)
          else:
            wait_scatter(u - 1)
          fire_scatter(i, b)
          if with_deg:
            fire_deg(i)
          if u < 2:
            fire_gather(i + 2, (u + 2) % 3)
          else:
            @pl.when(i + 2 < PCH)
            def _():
              fire_gather(i + 2, 1)
        return carry

      lax.fori_loop(0, PCH // 3, tbody, 0)

      wait_gather(0)
      wait_scatter(2)
      fire_scatter(PCH - 1, 0)
      if with_deg:
        fire_deg(PCH - 1)

      wait_scatter(0)
      if with_deg:
        drain_deg()

    plsc.subcore_barrier()

    @pl.when(s < NS - 1)
    def _():
      pltpu.sync_copy(acc_sh.at[pl.ds(s * 624, 624)],
                      agg_out.at[c, pl.ds(s * 624, 624)])
    @pl.when(s == NS - 1)
    def _():
      pltpu.sync_copy(acc_sh.at[pl.ds(9360, 640)],
                      agg_out.at[c, pl.ds(9360, 640)])
    if with_deg:
      @pl.when(s < NS - 1)
      def _():
        pltpu.sync_copy(deg_sh.at[pl.ds(s * 624, 624)],
                        degbuf.at[pl.ds(0, 624)])
        pltpu.sync_copy(degbuf.at[pl.ds(0, 624)],
                        deg_out.at[pl.ds(c * N + s * 624, 624)])
      @pl.when(s == NS - 1)
      def _():
        pltpu.sync_copy(deg_sh.at[pl.ds(9360, 640)],
                        degbuf.at[pl.ds(0, 640)])
        pltpu.sync_copy(degbuf.at[pl.ds(0, 640)],
                        deg_out.at[pl.ds(c * N + 9360, 640)])

  return pl.kernel(body, out_type=out_type, mesh=mesh,
                   scratch_types=scratch)


_sc_agg_deg = _make_sc_agg(D_IN, with_deg=True)
_sc_agg_p = _make_sc_agg(CP, with_deg=False)

R = 1000


def _tc1_body(a0, a1, d0, d1, x, w1, b1, w2, p_out):
  num = a0[0] + a1[0] + x[...]
  den = d0[0] + d1[0] + 1.0
  h = num / den
  h = jnp.maximum(jnp.dot(h, w1[...], preferred_element_type=jnp.float32)
                  + b1[...], 0.0)
  p_out[...] = jnp.dot(h, w2[...], preferred_element_type=jnp.float32)


def _tc2_body(g0, g1, d0, d1, p, b2, out):
  den = d0[0] + d1[0] + 1.0
  t = (g0[0] + g1[0] + p[...]) / den + b2[...]
  out[...] = t[:, :C]


def kernel(x, edge_index, W1, b1, W2, b2):
  src3 = edge_index[0].astype(jnp.int32).reshape(NW * PH, PCH, K)
  dst3 = edge_index[1].astype(jnp.int32).reshape(NW * PH, PCH, K)
  z2 = jnp.zeros((N, D_IN), jnp.float32)
  w2p = jnp.pad(W2, ((0, 0), (0, CP - C)))
  b2p = jnp.pad(b2, (0, CP - C)).reshape(1, CP)

  aggp, degp = _sc_agg_deg(x, src3, dst3, z2)
  degp3 = degp.reshape(NC, N, 1)

  grid = (N // R,)
  p = pl.pallas_call(
      _tc1_body,
      grid=grid,
      in_specs=[
          pl.BlockSpec((1, R, D_IN), lambda i: (0, i, 0)),
          pl.BlockSpec((1, R, D_IN), lambda i: (1, i, 0)),
          pl.BlockSpec((1, R, 1), lambda i: (0, i, 0)),
          pl.BlockSpec((1, R, 1), lambda i: (1, i, 0)),
          pl.BlockSpec((R, D_IN), lambda i: (i, 0)),
          pl.BlockSpec((D_IN, D_HID), lambda i: (0, 0)),
          pl.BlockSpec((1, D_HID), lambda i: (0, 0)),
          pl.BlockSpec((D_HID, CP), lambda i: (0, 0)),
      ],
      out_specs=pl.BlockSpec((R, CP), lambda i: (i, 0)),
      out_shape=jax.ShapeDtypeStruct((N, CP), jnp.float32),
  )(aggp, aggp, degp3, degp3, x, W1, b1.reshape(1, D_HID), w2p)

  (gp,) = _sc_agg_p(p, src3, dst3, z2)

  out = pl.pallas_call(
      _tc2_body,
      grid=grid,
      in_specs=[
          pl.BlockSpec((1, R, CP), lambda i: (0, i, 0)),
          pl.BlockSpec((1, R, CP), lambda i: (1, i, 0)),
          pl.BlockSpec((1, R, 1), lambda i: (0, i, 0)),
          pl.BlockSpec((1, R, 1), lambda i: (1, i, 0)),
          pl.BlockSpec((R, CP), lambda i: (i, 0)),
          pl.BlockSpec((1, CP), lambda i: (0, 0)),
      ],
      out_specs=pl.BlockSpec((R, C), lambda i: (i, 0)),
      out_shape=jax.ShapeDtypeStruct((N, C), jnp.float32),
  )(gp, gp, degp3, degp3, p, b2p)

  return out
